# Initial kernel scaffold; baseline (speedup 1.0000x reference)
import jax
import jax.numpy as jnp
from jax.experimental import pallas as pl


def kernel(x, node_attr, edge_attr, edge_index, node_W, node_b, edge_W, edge_b, conv_Wf, conv_bf, conv_Ws, conv_bs, conv_g, conv_be, sbn_g, sbn_b, fc_W, fc_b, fbn_g, fbn_b, head_W, head_b, head_g, head_be, out_W, out_b):
    raise NotImplementedError("write your pallas kernel here")


# speedup vs baseline: 1.7677x; 1.7677x over previous
import functools

import jax
import jax.numpy as jnp
from jax import lax
from jax.experimental import pallas as pl
from jax.experimental.pallas import tpu as pltpu
from jax.experimental.pallas import tpu_sc as plsc

EPS = 1e-5
NC = 2
NS = 16
F32 = jnp.float32


@functools.lru_cache(maxsize=None)
def _make_gather(N, E, D, K):
    NW = NC * NS
    chunk = E // NW
    iters = chunk // K
    assert chunk % K == 0 and E % NW == 0
    mesh = plsc.VectorSubcoreMesh(core_axis_name="c", subcore_axis_name="s")

    @functools.partial(
        pl.kernel,
        out_type=(jax.ShapeDtypeStruct((E, D), F32),
                  jax.ShapeDtypeStruct((E, D), F32)),
        mesh=mesh,
        scratch_types=[
            pltpu.VMEM((K,), jnp.int32),
            pltpu.VMEM((K,), jnp.int32),
            pltpu.VMEM((K, D), F32),
            pltpu.VMEM((K, D), F32),
            pltpu.SemaphoreType.DMA,
            pltpu.SemaphoreType.DMA,
        ],
    )
    def gather_k(table, dst_i, src_i, gd, gs, idx_d, idx_s, rows_d, rows_s,
                 sem_d, sem_s):
        wid = lax.axis_index("s") * NC + lax.axis_index("c")
        base0 = wid * chunk

        def body(i, carry):
            base = base0 + i * K
            pltpu.sync_copy(dst_i.at[pl.ds(base, K)], idx_d)
            pltpu.sync_copy(src_i.at[pl.ds(base, K)], idx_s)
            cd = pltpu.async_copy(table.at[idx_d], rows_d, sem_d)
            cs = pltpu.async_copy(table.at[idx_s], rows_s, sem_s)
            cd.wait()
            pltpu.sync_copy(rows_d, gd.at[pl.ds(base, K)])
            cs.wait()
            pltpu.sync_copy(rows_s, gs.at[pl.ds(base, K)])
            return carry

        lax.fori_loop(0, iters, body, 0)

    return gather_k


@functools.lru_cache(maxsize=None)
def _make_scatter(half, Hp, E, D, K):
    chunk = E // NS
    iters = chunk // K
    rpt = Hp // NS
    assert chunk % K == 0 and K % 16 == 0 and rpt % 8 == 0 and Hp > half
    mesh = plsc.VectorSubcoreMesh(core_axis_name="c", subcore_axis_name="s")

    @functools.partial(
        pl.kernel,
        out_type=jax.ShapeDtypeStruct((NC, Hp, D), F32),
        mesh=mesh,
        scratch_types=[
            pltpu.VMEM((K,), jnp.int32),
            pltpu.VMEM((K,), jnp.int32),
            pltpu.VMEM((K, D), F32),
            pltpu.VMEM((rpt, D), F32),
            pltpu.VMEM_SHARED((Hp, D), F32),
        ],
    )
    def scatter_k(msg, dst_i, zeros, part, idx_v, idx2_v, rows_v, big_v, aggr):
        cid = lax.axis_index("c")
        sid = lax.axis_index("s")
        lo = cid * half
        pltpu.sync_copy(zeros, big_v)
        pltpu.sync_copy(big_v, aggr.at[pl.ds(sid * rpt, rpt)])
        plsc.subcore_barrier()

        base0 = sid * chunk

        def body(i, carry):
            base = base0 + i * K
            pltpu.sync_copy(dst_i.at[pl.ds(base, K)], idx_v)
            pltpu.sync_copy(msg.at[pl.ds(base, K)], rows_v)
            for j in range(K // 16):
                v = idx_v[pl.ds(j * 16, 16)] - lo
                ok = jnp.logical_and(v >= 0, v < half)
                idx2_v[pl.ds(j * 16, 16)] = jnp.where(ok, v, half)
            pltpu.sync_copy(rows_v, aggr.at[idx2_v], add=True)
            return carry

        lax.fori_loop(0, iters, body, 0)
        plsc.subcore_barrier()
        pltpu.sync_copy(aggr.at[pl.ds(sid * rpt, rpt)], big_v)
        pltpu.sync_copy(big_v, part.at[cid, pl.ds(sid * rpt, rpt)])

    return scatter_k


def _h0_body(xb, nab, W, b, out):
    xa = jnp.concatenate([xb[...], nab[...]], axis=1)
    out[...] = jnp.dot(xa, W[...], preferred_element_type=F32) + b[...]


def _msg_body(gd, gs, eattr, eW, eb, Wfull, bias, out, *, ND):
    ea = jnp.dot(eattr[...], eW[...], preferred_element_type=F32) + eb[...]
    z = jnp.concatenate([gd[...], gs[...], ea], axis=1)
    pre = jnp.dot(z, Wfull[...], preferred_element_type=F32) + bias[...]
    a = pre[:, :ND]
    b = pre[:, ND:]
    out[...] = jax.nn.sigmoid(a) * jax.nn.softplus(b)


def _bn_body(part, h, g, be, out, s_acc, ss_acc, *, inv_n):
    ph = pl.program_id(0)
    i = pl.program_id(1)

    @pl.when(jnp.logical_and(ph == 0, i == 0))
    def _():
        s_acc[...] = jnp.zeros_like(s_acc)
        ss_acc[...] = jnp.zeros_like(ss_acc)

    aggr = part[0]

    @pl.when(ph == 0)
    def _():
        s_acc[...] += jnp.sum(aggr, axis=0, keepdims=True)

    @pl.when(ph == 1)
    def _():
        d = aggr - s_acc[...] * inv_n
        ss_acc[...] += jnp.sum(d * d, axis=0, keepdims=True)

    @pl.when(ph == 2)
    def _():
        m = s_acc[...] * inv_n
        v = ss_acc[...] * inv_n
        scale = g[...] * lax.rsqrt(v + EPS)
        out[...] = h[...] + (aggr - m) * scale + be[...]


def _stage_body(h, W, b, g, be, out, y_buf, s_acc, ss_acc, *, use_mm, Bn,
                inv_n):
    ph = pl.program_id(0)
    i = pl.program_id(1)

    @pl.when(jnp.logical_and(ph == 0, i == 0))
    def _():
        s_acc[...] = jnp.zeros_like(s_acc)
        ss_acc[...] = jnp.zeros_like(ss_acc)

    @pl.when(ph == 0)
    def _():
        if use_mm:
            y = jnp.dot(h[...], W[...], preferred_element_type=F32) + b[...]
        else:
            y = h[...]
        y_buf[pl.ds(i * Bn, Bn), :] = y
        s_acc[...] += jnp.sum(y, axis=0, keepdims=True)

    @pl.when(ph == 1)
    def _():
        d = y_buf[pl.ds(i * Bn, Bn), :] - s_acc[...] * inv_n
        ss_acc[...] += jnp.sum(d * d, axis=0, keepdims=True)

    @pl.when(ph == 2)
    def _():
        y = y_buf[pl.ds(i * Bn, Bn), :]
        m = s_acc[...] * inv_n
        v = ss_acc[...] * inv_n
        out[...] = jax.nn.softplus(
            g[...] * (y - m) * lax.rsqrt(v + EPS) + be[...])


def _final_body(h, W, b, out):
    out[...] = jnp.dot(h[...], W[...], preferred_element_type=F32) + b[...]


def _full(shape):
    return pl.BlockSpec(shape, lambda *args: (0,) * len(shape))


def kernel(x, node_attr, edge_attr, edge_index, node_W, node_b, edge_W,
           edge_b, conv_Wf, conv_bf, conv_Ws, conv_bs, conv_g, conv_be,
           sbn_g, sbn_b, fc_W, fc_b, fbn_g, fbn_b, head_W, head_b, head_g,
           head_be, out_W, out_b):
    N = x.shape[0]
    E = edge_attr.shape[0]
    ND = node_W.shape[1]
    ED = edge_W.shape[1]
    NL = conv_Wf.shape[0]
    NFC = head_W.shape[0]
    H = fc_W.shape[1]
    Bn = 1000
    Be = 2000
    K = 80
    nb = N // Bn
    inv_n = 1.0 / N

    src = edge_index[0]
    dst = edge_index[1]

    h = pl.pallas_call(
        _h0_body,
        grid=(nb,),
        in_specs=[
            pl.BlockSpec((Bn, 1), lambda i: (i, 0)),
            pl.BlockSpec((Bn, 2), lambda i: (i, 0)),
            _full((3, ND)),
            _full((1, ND)),
        ],
        out_specs=pl.BlockSpec((Bn, ND), lambda i: (i, 0)),
        out_shape=jax.ShapeDtypeStruct((N, ND), F32),
    )(x.reshape(N, 1), node_attr, node_W, node_b.reshape(1, ND))

    half = N // 2
    Hp = (half // (NS * 8) + 1) * (NS * 8)
    bpc = half // Bn
    gather_k = _make_gather(N, E, ND, K)
    scatter_k = _make_scatter(half, Hp, E, ND, K)
    zeros_init = jnp.zeros((Hp // NS, ND), F32)

    ZD = 2 * ND + ED
    msg_call = pl.pallas_call(
        functools.partial(_msg_body, ND=ND),
        grid=(E // Be,),
        in_specs=[
            pl.BlockSpec((Be, ND), lambda i: (i, 0)),
            pl.BlockSpec((Be, ND), lambda i: (i, 0)),
            pl.BlockSpec((Be, 2), lambda i: (i, 0)),
            _full((2, ED)),
            _full((1, ED)),
            _full((ZD, 2 * ND)),
            _full((1, 2 * ND)),
        ],
        out_specs=pl.BlockSpec((Be, ND), lambda i: (i, 0)),
        out_shape=jax.ShapeDtypeStruct((E, ND), F32),
    )
    eb2 = edge_b.reshape(1, ED)
    bias_all = jnp.concatenate([conv_bf, conv_bs], axis=1).reshape(NL, 1,
                                                                   2 * ND)
    Wfull_all = jnp.concatenate([conv_Wf, conv_Ws], axis=2)

    bn_call = pl.pallas_call(
        functools.partial(_bn_body, inv_n=inv_n),
        grid=(3, nb),
        in_specs=[
            pl.BlockSpec((1, Bn, ND), lambda p, i: (i // bpc, i % bpc, 0)),
            pl.BlockSpec((Bn, ND), lambda p, i: (i, 0)),
            _full((1, ND)),
            _full((1, ND)),
        ],
        out_specs=pl.BlockSpec((Bn, ND), lambda p, i: (i, 0)),
        out_shape=jax.ShapeDtypeStruct((N, ND), F32),
        scratch_shapes=[pltpu.VMEM((1, ND), F32), pltpu.VMEM((1, ND), F32)],
    )

    for l in range(NL):
        gd, gs = gather_k(h, dst, src)
        msg = msg_call(gd, gs, edge_attr, edge_W, eb2, Wfull_all[l],
                       bias_all[l])
        part = scatter_k(msg, dst, zeros_init)
        h = bn_call(part, h, conv_g[l].reshape(1, ND), conv_be[l].reshape(1, ND))

    def stage(h, W, b, g, be, use_mm, Dout):
        return pl.pallas_call(
            functools.partial(_stage_body, use_mm=use_mm, Bn=Bn, inv_n=inv_n),
            grid=(3, nb),
            in_specs=[
                pl.BlockSpec((Bn, ND), lambda p, i: (i, 0)),
                _full((ND, Dout)),
                _full((1, Dout)),
                _full((1, Dout)),
                _full((1, Dout)),
            ],
            out_specs=pl.BlockSpec((Bn, Dout), lambda p, i: (i, 0)),
            out_shape=jax.ShapeDtypeStruct((N, Dout), F32),
            scratch_shapes=[
                pltpu.VMEM((N, Dout), F32),
                pltpu.VMEM((1, Dout), F32),
                pltpu.VMEM((1, Dout), F32),
            ],
        )(h, W, b.reshape(1, Dout), g.reshape(1, Dout), be.reshape(1, Dout))

    eye = jnp.eye(ND, dtype=F32)
    h = stage(h, eye, jnp.zeros((ND,), F32), sbn_g, sbn_b, False, ND)
    h = stage(h, fc_W, fc_b, fbn_g, fbn_b, True, H)
    for i in range(NFC):
        h = stage(h, head_W[i], head_b[i], head_g[i], head_be[i], True, H)

    out = pl.pallas_call(
        _final_body,
        grid=(nb,),
        in_specs=[
            pl.BlockSpec((Bn, H), lambda i: (i, 0)),
            _full((H, 1)),
            _full((1, 1)),
        ],
        out_specs=pl.BlockSpec((Bn, 1), lambda i: (i, 0)),
        out_shape=jax.ShapeDtypeStruct((N, 1), F32),
    )(h, out_W, out_b.reshape(1, 1))
    return out


# speedup vs baseline: 2.5252x; 1.4285x over previous
import functools

import jax
import jax.numpy as jnp
from jax import lax
from jax.experimental import pallas as pl
from jax.experimental.pallas import tpu as pltpu
from jax.experimental.pallas import tpu_sc as plsc

EPS = 1e-5
NC = 2
NS = 16
F32 = jnp.float32


@functools.lru_cache(maxsize=None)
def _make_gather(N, E, D, K):
    NW = NC * NS
    chunk = E // NW
    iters = chunk // K
    assert chunk % K == 0 and E % NW == 0 and iters % 2 == 1 and iters >= 3
    mesh = plsc.VectorSubcoreMesh(core_axis_name="c", subcore_axis_name="s")

    @functools.partial(
        pl.kernel,
        out_type=(jax.ShapeDtypeStruct((E, D), F32),
                  jax.ShapeDtypeStruct((E, D), F32)),
        mesh=mesh,
        scratch_types=[
            pltpu.VMEM((chunk,), jnp.int32),
            pltpu.VMEM((chunk,), jnp.int32),
            [pltpu.VMEM((K, D), F32)] * 4,
            [pltpu.SemaphoreType.DMA] * 4,
        ],
    )
    def gather_k(table, dst_i, src_i, gd, gs, idx_d, idx_s, rows, sems):
        wid = lax.axis_index("s") * NC + lax.axis_index("c")
        base0 = wid * chunk
        pltpu.sync_copy(dst_i.at[pl.ds(base0, chunk)], idx_d)
        pltpu.sync_copy(src_i.at[pl.ds(base0, chunk)], idx_s)

        def start(j, b):
            off = j * K
            pltpu.async_copy(table.at[idx_d.at[pl.ds(off, K)]],
                             rows[2 * b], sems[2 * b])
            pltpu.async_copy(table.at[idx_s.at[pl.ds(off, K)]],
                             rows[2 * b + 1], sems[2 * b + 1])

        def finish(j, b):
            base = base0 + j * K
            pltpu.make_async_copy(table.at[idx_d.at[pl.ds(0, K)]],
                                  rows[2 * b], sems[2 * b]).wait()
            pltpu.sync_copy(rows[2 * b], gd.at[pl.ds(base, K)])
            pltpu.make_async_copy(table.at[idx_s.at[pl.ds(0, K)]],
                                  rows[2 * b + 1], sems[2 * b + 1]).wait()
            pltpu.sync_copy(rows[2 * b + 1], gs.at[pl.ds(base, K)])

        start(0, 0)

        def body(t, carry):
            j = 2 * t
            start(j + 1, 1)
            finish(j, 0)
            start(j + 2, 0)
            finish(j + 1, 1)
            return carry

        lax.fori_loop(0, (iters - 1) // 2, body, 0)
        finish(iters - 1, 0)

    return gather_k


@functools.lru_cache(maxsize=None)
def _make_scatter(half, Hp, E, D, K):
    chunk = E // NS
    iters = chunk // K
    rpt = Hp // NS
    assert chunk % K == 0 and K % 16 == 0 and rpt % 8 == 0 and Hp > half
    assert iters % 2 == 0
    mesh = plsc.VectorSubcoreMesh(core_axis_name="c", subcore_axis_name="s")

    @functools.partial(
        pl.kernel,
        out_type=jax.ShapeDtypeStruct((NC, Hp, D), F32),
        mesh=mesh,
        scratch_types=[
            pltpu.VMEM((chunk,), jnp.int32),
            [pltpu.VMEM((K,), jnp.int32)] * 2,
            [pltpu.VMEM((K, D), F32)] * 2,
            [pltpu.SemaphoreType.DMA] * 2,
            pltpu.VMEM((rpt, D), F32),
            pltpu.VMEM_SHARED((Hp, D), F32),
        ],
    )
    def scatter_k(msg, dst_i, zeros, part, idx_all, idx2, rows, sems, big_v,
                  aggr):
        cid = lax.axis_index("c")
        sid = lax.axis_index("s")
        lo = cid * half
        base0 = sid * chunk
        pltpu.sync_copy(zeros, big_v)
        pltpu.sync_copy(big_v, aggr.at[pl.ds(sid * rpt, rpt)])
        pltpu.sync_copy(dst_i.at[pl.ds(base0, chunk)], idx_all)
        plsc.subcore_barrier()

        def start(j, b):
            pltpu.async_copy(msg.at[pl.ds(base0 + j * K, K)], rows[b],
                             sems[b])

        def finish(j, b):
            off = j * K
            for t in range(K // 16):
                v = idx_all[pl.ds(off + t * 16, 16)] - lo
                ok = jnp.logical_and(v >= 0, v < half)
                idx2[b][pl.ds(t * 16, 16)] = jnp.where(ok, v, half)
            pltpu.make_async_copy(msg.at[pl.ds(0, K)], rows[b],
                                  sems[b]).wait()
            pltpu.sync_copy(rows[b], aggr.at[idx2[b]], add=True)

        start(0, 0)

        def body(t, carry):
            j = 2 * t
            start(j + 1, 1)
            finish(j, 0)

            @pl.when(j + 2 < iters)
            def _():
                start(j + 2, 0)

            finish(j + 1, 1)
            return carry

        lax.fori_loop(0, iters // 2, body, 0)
        plsc.subcore_barrier()
        pltpu.sync_copy(aggr.at[pl.ds(sid * rpt, rpt)], big_v)
        pltpu.sync_copy(big_v, part.at[cid, pl.ds(sid * rpt, rpt)])

    return scatter_k


def _h0_body(xb, nab, W, b, out):
    xa = jnp.concatenate([xb[...], nab[...]], axis=1)
    out[...] = jnp.dot(xa, W[...], preferred_element_type=F32) + b[...]


def _msg_body(gd, gs, eattr, eW, eb, Wfull, bias, out, *, ND):
    ea = jnp.dot(eattr[...], eW[...], preferred_element_type=F32) + eb[...]
    z = jnp.concatenate([gd[...], gs[...], ea], axis=1)
    pre = jnp.dot(z, Wfull[...], preferred_element_type=F32) + bias[...]
    a = pre[:, :ND]
    b = pre[:, ND:]
    out[...] = jax.nn.sigmoid(a) * jax.nn.softplus(b)


def _bn_body(part, h, g, be, out, s_acc, ss_acc, *, inv_n):
    ph = pl.program_id(0)
    i = pl.program_id(1)

    @pl.when(jnp.logical_and(ph == 0, i == 0))
    def _():
        s_acc[...] = jnp.zeros_like(s_acc)
        ss_acc[...] = jnp.zeros_like(ss_acc)

    aggr = part[0]

    @pl.when(ph == 0)
    def _():
        s_acc[...] += jnp.sum(aggr, axis=0, keepdims=True)

    @pl.when(ph == 1)
    def _():
        d = aggr - s_acc[...] * inv_n
        ss_acc[...] += jnp.sum(d * d, axis=0, keepdims=True)

    @pl.when(ph == 2)
    def _():
        m = s_acc[...] * inv_n
        v = ss_acc[...] * inv_n
        scale = g[...] * lax.rsqrt(v + EPS)
        out[...] = h[...] + (aggr - m) * scale + be[...]


def _stage_body(h, W, b, g, be, out, y_buf, s_acc, ss_acc, *, use_mm, Bn,
                inv_n):
    ph = pl.program_id(0)
    i = pl.program_id(1)

    @pl.when(jnp.logical_and(ph == 0, i == 0))
    def _():
        s_acc[...] = jnp.zeros_like(s_acc)
        ss_acc[...] = jnp.zeros_like(ss_acc)

    @pl.when(ph == 0)
    def _():
        if use_mm:
            y = jnp.dot(h[...], W[...], preferred_element_type=F32) + b[...]
        else:
            y = h[...]
        y_buf[pl.ds(i * Bn, Bn), :] = y
        s_acc[...] += jnp.sum(y, axis=0, keepdims=True)

    @pl.when(ph == 1)
    def _():
        d = y_buf[pl.ds(i * Bn, Bn), :] - s_acc[...] * inv_n
        ss_acc[...] += jnp.sum(d * d, axis=0, keepdims=True)

    @pl.when(ph == 2)
    def _():
        y = y_buf[pl.ds(i * Bn, Bn), :]
        m = s_acc[...] * inv_n
        v = ss_acc[...] * inv_n
        out[...] = jax.nn.softplus(
            g[...] * (y - m) * lax.rsqrt(v + EPS) + be[...])


def _final_body(h, W, b, out):
    out[...] = jnp.dot(h[...], W[...], preferred_element_type=F32) + b[...]


def _full(shape):
    return pl.BlockSpec(shape, lambda *args: (0,) * len(shape))


def kernel(x, node_attr, edge_attr, edge_index, node_W, node_b, edge_W,
           edge_b, conv_Wf, conv_bf, conv_Ws, conv_bs, conv_g, conv_be,
           sbn_g, sbn_b, fc_W, fc_b, fbn_g, fbn_b, head_W, head_b, head_g,
           head_be, out_W, out_b):
    N = x.shape[0]
    E = edge_attr.shape[0]
    ND = node_W.shape[1]
    ED = edge_W.shape[1]
    NL = conv_Wf.shape[0]
    NFC = head_W.shape[0]
    H = fc_W.shape[1]
    Bn = 1000
    Be = 2000
    K = 80
    nb = N // Bn
    inv_n = 1.0 / N

    src = edge_index[0]
    dst = edge_index[1]

    h = pl.pallas_call(
        _h0_body,
        grid=(nb,),
        in_specs=[
            pl.BlockSpec((Bn, 1), lambda i: (i, 0)),
            pl.BlockSpec((Bn, 2), lambda i: (i, 0)),
            _full((3, ND)),
            _full((1, ND)),
        ],
        out_specs=pl.BlockSpec((Bn, ND), lambda i: (i, 0)),
        out_shape=jax.ShapeDtypeStruct((N, ND), F32),
    )(x.reshape(N, 1), node_attr, node_W, node_b.reshape(1, ND))

    half = N // 2
    Hp = (half // (NS * 8) + 1) * (NS * 8)
    bpc = half // Bn
    gather_k = _make_gather(N, E, ND, K)
    scatter_k = _make_scatter(half, Hp, E, ND, K)
    zeros_init = jnp.zeros((Hp // NS, ND), F32)

    ZD = 2 * ND + ED
    msg_call = pl.pallas_call(
        functools.partial(_msg_body, ND=ND),
        grid=(E // Be,),
        in_specs=[
            pl.BlockSpec((Be, ND), lambda i: (i, 0)),
            pl.BlockSpec((Be, ND), lambda i: (i, 0)),
            pl.BlockSpec((Be, 2), lambda i: (i, 0)),
            _full((2, ED)),
            _full((1, ED)),
            _full((ZD, 2 * ND)),
            _full((1, 2 * ND)),
        ],
        out_specs=pl.BlockSpec((Be, ND), lambda i: (i, 0)),
        out_shape=jax.ShapeDtypeStruct((E, ND), F32),
    )
    eb2 = edge_b.reshape(1, ED)
    bias_all = jnp.concatenate([conv_bf, conv_bs], axis=1).reshape(NL, 1,
                                                                   2 * ND)
    Wfull_all = jnp.concatenate([conv_Wf, conv_Ws], axis=2)

    bn_call = pl.pallas_call(
        functools.partial(_bn_body, inv_n=inv_n),
        grid=(3, nb),
        in_specs=[
            pl.BlockSpec((1, Bn, ND), lambda p, i: (i // bpc, i % bpc, 0)),
            pl.BlockSpec((Bn, ND), lambda p, i: (i, 0)),
            _full((1, ND)),
            _full((1, ND)),
        ],
        out_specs=pl.BlockSpec((Bn, ND), lambda p, i: (i, 0)),
        out_shape=jax.ShapeDtypeStruct((N, ND), F32),
        scratch_shapes=[pltpu.VMEM((1, ND), F32), pltpu.VMEM((1, ND), F32)],
    )

    for l in range(NL):
        gd, gs = gather_k(h, dst, src)
        msg = msg_call(gd, gs, edge_attr, edge_W, eb2, Wfull_all[l],
                       bias_all[l])
        part = scatter_k(msg, dst, zeros_init)
        h = bn_call(part, h, conv_g[l].reshape(1, ND), conv_be[l].reshape(1, ND))

    def stage(h, W, b, g, be, use_mm, Dout):
        return pl.pallas_call(
            functools.partial(_stage_body, use_mm=use_mm, Bn=Bn, inv_n=inv_n),
            grid=(3, nb),
            in_specs=[
                pl.BlockSpec((Bn, ND), lambda p, i: (i, 0)),
                _full((ND, Dout)),
                _full((1, Dout)),
                _full((1, Dout)),
                _full((1, Dout)),
            ],
            out_specs=pl.BlockSpec((Bn, Dout), lambda p, i: (i, 0)),
            out_shape=jax.ShapeDtypeStruct((N, Dout), F32),
            scratch_shapes=[
                pltpu.VMEM((N, Dout), F32),
                pltpu.VMEM((1, Dout), F32),
                pltpu.VMEM((1, Dout), F32),
            ],
        )(h, W, b.reshape(1, Dout), g.reshape(1, Dout), be.reshape(1, Dout))

    eye = jnp.eye(ND, dtype=F32)
    h = stage(h, eye, jnp.zeros((ND,), F32), sbn_g, sbn_b, False, ND)
    h = stage(h, fc_W, fc_b, fbn_g, fbn_b, True, H)
    for i in range(NFC):
        h = stage(h, head_W[i], head_b[i], head_g[i], head_be[i], True, H)

    out = pl.pallas_call(
        _final_body,
        grid=(nb,),
        in_specs=[
            pl.BlockSpec((Bn, H), lambda i: (i, 0)),
            _full((H, 1)),
            _full((1, 1)),
        ],
        out_specs=pl.BlockSpec((Bn, 1), lambda i: (i, 0)),
        out_shape=jax.ShapeDtypeStruct((N, 1), F32),
    )(h, out_W, out_b.reshape(1, 1))
    return out


# speedup vs baseline: 2.9128x; 1.1535x over previous
import functools

import jax
import jax.numpy as jnp
from jax import lax
from jax.experimental import pallas as pl
from jax.experimental.pallas import tpu as pltpu
from jax.experimental.pallas import tpu_sc as plsc

EPS = 1e-5
NC = 2
NS = 16
F32 = jnp.float32


@functools.lru_cache(maxsize=None)
def _make_gather(N, E, D, K, e0, ne):
    NW = NC * NS
    chunk = ne // NW
    iters = chunk // K
    assert chunk % K == 0 and ne % NW == 0 and iters >= 3 and e0 % 8 == 0
    mesh = plsc.VectorSubcoreMesh(core_axis_name="c", subcore_axis_name="s")

    @functools.partial(
        pl.kernel,
        out_type=(jax.ShapeDtypeStruct((ne, D), F32),
                  jax.ShapeDtypeStruct((ne, D), F32)),
        mesh=mesh,
        scratch_types=[
            pltpu.VMEM((chunk,), jnp.int32),
            pltpu.VMEM((chunk,), jnp.int32),
            [pltpu.VMEM((K, D), F32)] * 4,
            [pltpu.SemaphoreType.DMA] * 4,
        ],
    )
    def gather_k(table, dst_i, src_i, gd, gs, idx_d, idx_s, rows, sems):
        wid = lax.axis_index("s") * NC + lax.axis_index("c")
        base0 = wid * chunk
        pltpu.sync_copy(dst_i.at[pl.ds(e0 + base0, chunk)], idx_d)
        pltpu.sync_copy(src_i.at[pl.ds(e0 + base0, chunk)], idx_s)

        def start(j, b):
            off = j * K
            pltpu.async_copy(table.at[idx_d.at[pl.ds(off, K)]],
                             rows[2 * b], sems[2 * b])
            pltpu.async_copy(table.at[idx_s.at[pl.ds(off, K)]],
                             rows[2 * b + 1], sems[2 * b + 1])

        def finish(j, b):
            base = base0 + j * K
            pltpu.make_async_copy(table.at[idx_d.at[pl.ds(0, K)]],
                                  rows[2 * b], sems[2 * b]).wait()
            pltpu.sync_copy(rows[2 * b], gd.at[pl.ds(base, K)])
            pltpu.make_async_copy(table.at[idx_s.at[pl.ds(0, K)]],
                                  rows[2 * b + 1], sems[2 * b + 1]).wait()
            pltpu.sync_copy(rows[2 * b + 1], gs.at[pl.ds(base, K)])

        start(0, 0)

        def body(t, carry):
            j = 2 * t
            start(j + 1, 1)
            finish(j, 0)

            @pl.when(j + 2 < iters)
            def _():
                start(j + 2, 0)

            finish(j + 1, 1)
            return carry

        lax.fori_loop(0, iters // 2, body, 0)
        if iters % 2 == 1:
            finish(iters - 1, 0)

    return gather_k


@functools.lru_cache(maxsize=None)
def _make_scatter(half, Hp, E, D, K, e0, ne):
    chunk = ne // NS
    iters = chunk // K
    rpt = Hp // NS
    assert chunk % K == 0 and K % 16 == 0 and rpt % 8 == 0 and Hp > half
    assert iters >= 3 and e0 % 8 == 0
    mesh = plsc.VectorSubcoreMesh(core_axis_name="c", subcore_axis_name="s")

    @functools.partial(
        pl.kernel,
        out_type=jax.ShapeDtypeStruct((NC, Hp, D), F32),
        mesh=mesh,
        scratch_types=[
            pltpu.VMEM((chunk,), jnp.int32),
            [pltpu.VMEM((K,), jnp.int32)] * 2,
            [pltpu.VMEM((K, D), F32)] * 2,
            [pltpu.SemaphoreType.DMA] * 2,
            pltpu.VMEM((rpt, D), F32),
            pltpu.VMEM_SHARED((Hp, D), F32),
        ],
    )
    def scatter_k(msg, dst_i, zeros, part, idx_all, idx2, rows, sems, big_v,
                  aggr):
        cid = lax.axis_index("c")
        sid = lax.axis_index("s")
        lo = cid * half
        base0 = sid * chunk
        pltpu.sync_copy(zeros, big_v)
        pltpu.sync_copy(big_v, aggr.at[pl.ds(sid * rpt, rpt)])
        pltpu.sync_copy(dst_i.at[pl.ds(e0 + base0, chunk)], idx_all)
        plsc.subcore_barrier()

        def start(j, b):
            pltpu.async_copy(msg.at[pl.ds(base0 + j * K, K)], rows[b],
                             sems[b])

        def finish(j, b):
            off = j * K
            for t in range(K // 16):
                v = idx_all[pl.ds(off + t * 16, 16)] - lo
                ok = jnp.logical_and(v >= 0, v < half)
                idx2[b][pl.ds(t * 16, 16)] = jnp.where(ok, v, half)
            pltpu.make_async_copy(msg.at[pl.ds(0, K)], rows[b],
                                  sems[b]).wait()
            pltpu.sync_copy(rows[b], aggr.at[idx2[b]], add=True)

        start(0, 0)

        def body(t, carry):
            j = 2 * t
            start(j + 1, 1)
            finish(j, 0)

            @pl.when(j + 2 < iters)
            def _():
                start(j + 2, 0)

            finish(j + 1, 1)
            return carry

        lax.fori_loop(0, iters // 2, body, 0)
        if iters % 2 == 1:
            finish(iters - 1, 0)
        plsc.subcore_barrier()
        pltpu.sync_copy(aggr.at[pl.ds(sid * rpt, rpt)], big_v)
        pltpu.sync_copy(big_v, part.at[cid, pl.ds(sid * rpt, rpt)])

    return scatter_k


def _h0_body(xb, nab, W, b, out):
    xa = jnp.concatenate([xb[...], nab[...]], axis=1)
    out[...] = jnp.dot(xa, W[...], preferred_element_type=F32) + b[...]


def _msg_body(gd, gs, eattr, eW, eb, Wfull, bias, out, *, ND):
    ea = jnp.dot(eattr[...], eW[...], preferred_element_type=F32) + eb[...]
    z = jnp.concatenate([gd[...], gs[...], ea], axis=1)
    pre = jnp.dot(z, Wfull[...], preferred_element_type=F32) + bias[...]
    a = pre[:, :ND]
    b = pre[:, ND:]
    out[...] = jax.nn.sigmoid(a) * jax.nn.softplus(b)


def _bn_body(pa, pb, h, g, be, out, s_acc, ss_acc, *, inv_n):
    ph = pl.program_id(0)
    i = pl.program_id(1)

    @pl.when(jnp.logical_and(ph == 0, i == 0))
    def _():
        s_acc[...] = jnp.zeros_like(s_acc)
        ss_acc[...] = jnp.zeros_like(ss_acc)

    aggr = pa[0] + pb[0]

    @pl.when(ph == 0)
    def _():
        s_acc[...] += jnp.sum(aggr, axis=0, keepdims=True)

    @pl.when(ph == 1)
    def _():
        d = aggr - s_acc[...] * inv_n
        ss_acc[...] += jnp.sum(d * d, axis=0, keepdims=True)

    @pl.when(ph == 2)
    def _():
        m = s_acc[...] * inv_n
        v = ss_acc[...] * inv_n
        scale = g[...] * lax.rsqrt(v + EPS)
        out[...] = h[...] + (aggr - m) * scale + be[...]


def _stage_body(h, W, b, g, be, out, y_buf, s_acc, ss_acc, *, use_mm, Bn,
                inv_n):
    ph = pl.program_id(0)
    i = pl.program_id(1)

    @pl.when(jnp.logical_and(ph == 0, i == 0))
    def _():
        s_acc[...] = jnp.zeros_like(s_acc)
        ss_acc[...] = jnp.zeros_like(ss_acc)

    @pl.when(ph == 0)
    def _():
        if use_mm:
            y = jnp.dot(h[...], W[...], preferred_element_type=F32) + b[...]
        else:
            y = h[...]
        y_buf[pl.ds(i * Bn, Bn), :] = y
        s_acc[...] += jnp.sum(y, axis=0, keepdims=True)

    @pl.when(ph == 1)
    def _():
        d = y_buf[pl.ds(i * Bn, Bn), :] - s_acc[...] * inv_n
        ss_acc[...] += jnp.sum(d * d, axis=0, keepdims=True)

    @pl.when(ph == 2)
    def _():
        y = y_buf[pl.ds(i * Bn, Bn), :]
        m = s_acc[...] * inv_n
        v = ss_acc[...] * inv_n
        out[...] = jax.nn.softplus(
            g[...] * (y - m) * lax.rsqrt(v + EPS) + be[...])


def _final_body(h, W, b, out):
    out[...] = jnp.dot(h[...], W[...], preferred_element_type=F32) + b[...]


def _full(shape):
    return pl.BlockSpec(shape, lambda *args: (0,) * len(shape))


def kernel(x, node_attr, edge_attr, edge_index, node_W, node_b, edge_W,
           edge_b, conv_Wf, conv_bf, conv_Ws, conv_bs, conv_g, conv_be,
           sbn_g, sbn_b, fc_W, fc_b, fbn_g, fbn_b, head_W, head_b, head_g,
           head_be, out_W, out_b):
    N = x.shape[0]
    E = edge_attr.shape[0]
    ND = node_W.shape[1]
    ED = edge_W.shape[1]
    NL = conv_Wf.shape[0]
    NFC = head_W.shape[0]
    H = fc_W.shape[1]
    Bn = 1000
    Be = 2000
    K = 80
    nb = N // Bn
    inv_n = 1.0 / N

    src = edge_index[0]
    dst = edge_index[1]

    h = pl.pallas_call(
        _h0_body,
        grid=(nb,),
        in_specs=[
            pl.BlockSpec((Bn, 1), lambda i: (i, 0)),
            pl.BlockSpec((Bn, 2), lambda i: (i, 0)),
            _full((3, ND)),
            _full((1, ND)),
        ],
        out_specs=pl.BlockSpec((Bn, ND), lambda i: (i, 0)),
        out_shape=jax.ShapeDtypeStruct((N, ND), F32),
    )(x.reshape(N, 1), node_attr, node_W, node_b.reshape(1, ND))

    half = N // 2
    Hp = (half // (NS * 8) + 1) * (NS * 8)
    bpc = half // Bn
    ne = E // 2
    Kg = 40
    chunks = [(0, ne), (ne, ne)]
    gathers = [_make_gather(N, E, ND, Kg, e0, n_) for e0, n_ in chunks]
    scatters = [_make_scatter(half, Hp, E, ND, K, e0, n_) for e0, n_ in chunks]
    zeros_init = jnp.zeros((Hp // NS, ND), F32)

    ZD = 2 * ND + ED

    def make_msg_call(e0, n_):
        blk0 = e0 // Be
        return pl.pallas_call(
            functools.partial(_msg_body, ND=ND),
            grid=(n_ // Be,),
            in_specs=[
                pl.BlockSpec((Be, ND), lambda i: (i, 0)),
                pl.BlockSpec((Be, ND), lambda i: (i, 0)),
                pl.BlockSpec((Be, 2), lambda i: (i + blk0, 0)),
                _full((2, ED)),
                _full((1, ED)),
                _full((ZD, 2 * ND)),
                _full((1, 2 * ND)),
            ],
            out_specs=pl.BlockSpec((Be, ND), lambda i: (i, 0)),
            out_shape=jax.ShapeDtypeStruct((n_, ND), F32),
        )

    msg_calls = [make_msg_call(e0, n_) for e0, n_ in chunks]
    eb2 = edge_b.reshape(1, ED)
    bias_all = jnp.concatenate([conv_bf, conv_bs], axis=1).reshape(NL, 1,
                                                                   2 * ND)
    Wfull_all = jnp.concatenate([conv_Wf, conv_Ws], axis=2)

    part_spec = pl.BlockSpec((1, Bn, ND), lambda p, i: (i // bpc, i % bpc, 0))
    bn_call = pl.pallas_call(
        functools.partial(_bn_body, inv_n=inv_n),
        grid=(3, nb),
        in_specs=[
            part_spec,
            part_spec,
            pl.BlockSpec((Bn, ND), lambda p, i: (i, 0)),
            _full((1, ND)),
            _full((1, ND)),
        ],
        out_specs=pl.BlockSpec((Bn, ND), lambda p, i: (i, 0)),
        out_shape=jax.ShapeDtypeStruct((N, ND), F32),
        scratch_shapes=[pltpu.VMEM((1, ND), F32), pltpu.VMEM((1, ND), F32)],
    )

    for l in range(NL):
        gs_pairs = [g_k(h, dst, src) for g_k in gathers]
        msgs = [m_c(gd, gs, edge_attr, edge_W, eb2, Wfull_all[l], bias_all[l])
                for m_c, (gd, gs) in zip(msg_calls, gs_pairs)]
        parts = [s_k(m, dst, zeros_init) for s_k, m in zip(scatters, msgs)]
        h = bn_call(parts[0], parts[1], h, conv_g[l].reshape(1, ND),
                    conv_be[l].reshape(1, ND))

    def stage(h, W, b, g, be, use_mm, Dout):
        return pl.pallas_call(
            functools.partial(_stage_body, use_mm=use_mm, Bn=Bn, inv_n=inv_n),
            grid=(3, nb),
            in_specs=[
                pl.BlockSpec((Bn, ND), lambda p, i: (i, 0)),
                _full((ND, Dout)),
                _full((1, Dout)),
                _full((1, Dout)),
                _full((1, Dout)),
            ],
            out_specs=pl.BlockSpec((Bn, Dout), lambda p, i: (i, 0)),
            out_shape=jax.ShapeDtypeStruct((N, Dout), F32),
            scratch_shapes=[
                pltpu.VMEM((N, Dout), F32),
                pltpu.VMEM((1, Dout), F32),
                pltpu.VMEM((1, Dout), F32),
            ],
        )(h, W, b.reshape(1, Dout), g.reshape(1, Dout), be.reshape(1, Dout))

    eye = jnp.eye(ND, dtype=F32)
    h = stage(h, eye, jnp.zeros((ND,), F32), sbn_g, sbn_b, False, ND)
    h = stage(h, fc_W, fc_b, fbn_g, fbn_b, True, H)
    for i in range(NFC):
        h = stage(h, head_W[i], head_b[i], head_g[i], head_be[i], True, H)

    out = pl.pallas_call(
        _final_body,
        grid=(nb,),
        in_specs=[
            pl.BlockSpec((Bn, H), lambda i: (i, 0)),
            _full((H, 1)),
            _full((1, 1)),
        ],
        out_specs=pl.BlockSpec((Bn, 1), lambda i: (i, 0)),
        out_shape=jax.ShapeDtypeStruct((N, 1), F32),
    )(h, out_W, out_b.reshape(1, 1))
    return out


# speedup vs baseline: 2.9559x; 1.0148x over previous
import functools

import jax
import jax.numpy as jnp
from jax import lax
from jax.experimental import pallas as pl
from jax.experimental.pallas import tpu as pltpu
from jax.experimental.pallas import tpu_sc as plsc

EPS = 1e-5
NC = 2
NS = 16
F32 = jnp.float32


@functools.lru_cache(maxsize=None)
def _make_gather(N, E, D, K, e0, ne):
    NW = NC * NS
    chunk = ne // NW
    iters = chunk // K
    assert chunk % K == 0 and ne % NW == 0 and iters >= 3 and e0 % 8 == 0
    mesh = plsc.VectorSubcoreMesh(core_axis_name="c", subcore_axis_name="s")

    @functools.partial(
        pl.kernel,
        out_type=(jax.ShapeDtypeStruct((ne, D), F32),
                  jax.ShapeDtypeStruct((ne, D), F32)),
        mesh=mesh,
        scratch_types=[
            pltpu.VMEM((chunk,), jnp.int32),
            pltpu.VMEM((chunk,), jnp.int32),
            [pltpu.VMEM((K, D), F32)] * 4,
            [pltpu.SemaphoreType.DMA] * 4,
        ],
    )
    def gather_k(table, dst_i, src_i, gd, gs, idx_d, idx_s, rows, sems):
        wid = lax.axis_index("s") * NC + lax.axis_index("c")
        base0 = wid * chunk
        pltpu.sync_copy(dst_i.at[pl.ds(e0 + base0, chunk)], idx_d)
        pltpu.sync_copy(src_i.at[pl.ds(e0 + base0, chunk)], idx_s)

        def start(j, b):
            off = j * K
            pltpu.async_copy(table.at[idx_d.at[pl.ds(off, K)]],
                             rows[2 * b], sems[2 * b])
            pltpu.async_copy(table.at[idx_s.at[pl.ds(off, K)]],
                             rows[2 * b + 1], sems[2 * b + 1])

        def finish(j, b):
            base = base0 + j * K
            pltpu.make_async_copy(table.at[idx_d.at[pl.ds(0, K)]],
                                  rows[2 * b], sems[2 * b]).wait()
            pltpu.sync_copy(rows[2 * b], gd.at[pl.ds(base, K)])
            pltpu.make_async_copy(table.at[idx_s.at[pl.ds(0, K)]],
                                  rows[2 * b + 1], sems[2 * b + 1]).wait()
            pltpu.sync_copy(rows[2 * b + 1], gs.at[pl.ds(base, K)])

        start(0, 0)

        def body(t, carry):
            j = 2 * t
            start(j + 1, 1)
            finish(j, 0)

            @pl.when(j + 2 < iters)
            def _():
                start(j + 2, 0)

            finish(j + 1, 1)
            return carry

        lax.fori_loop(0, iters // 2, body, 0)
        if iters % 2 == 1:
            finish(iters - 1, 0)

    return gather_k


@functools.lru_cache(maxsize=None)
def _make_scatter(half, Hp, E, D, K, e0, ne):
    chunk = ne // NS
    iters = chunk // K
    rpt = Hp // NS
    assert chunk % K == 0 and K % 16 == 0 and rpt % 8 == 0 and Hp > half
    assert iters >= 3 and e0 % 8 == 0
    mesh = plsc.VectorSubcoreMesh(core_axis_name="c", subcore_axis_name="s")

    @functools.partial(
        pl.kernel,
        out_type=jax.ShapeDtypeStruct((NC, Hp, D), F32),
        mesh=mesh,
        scratch_types=[
            pltpu.VMEM((chunk,), jnp.int32),
            [pltpu.VMEM((K,), jnp.int32)] * 2,
            [pltpu.VMEM((K, D), F32)] * 2,
            [pltpu.SemaphoreType.DMA] * 2,
            pltpu.VMEM((rpt, D), F32),
            pltpu.VMEM_SHARED((Hp, D), F32),
        ],
    )
    def scatter_k(msg, dst_i, zeros, part, idx_all, idx2, rows, sems, big_v,
                  aggr):
        cid = lax.axis_index("c")
        sid = lax.axis_index("s")
        lo = cid * half
        base0 = sid * chunk
        pltpu.sync_copy(zeros, big_v)
        pltpu.sync_copy(big_v, aggr.at[pl.ds(sid * rpt, rpt)])
        pltpu.sync_copy(dst_i.at[pl.ds(e0 + base0, chunk)], idx_all)
        plsc.subcore_barrier()

        def start(j, b):
            pltpu.async_copy(msg.at[pl.ds(base0 + j * K, K)], rows[b],
                             sems[b])

        def finish(j, b):
            off = j * K
            for t in range(K // 16):
                v = idx_all[pl.ds(off + t * 16, 16)] - lo
                ok = jnp.logical_and(v >= 0, v < half)
                idx2[b][pl.ds(t * 16, 16)] = jnp.where(ok, v, half)
            pltpu.make_async_copy(msg.at[pl.ds(0, K)], rows[b],
                                  sems[b]).wait()
            pltpu.sync_copy(rows[b], aggr.at[idx2[b]], add=True)

        start(0, 0)

        def body(t, carry):
            j = 2 * t
            start(j + 1, 1)
            finish(j, 0)

            @pl.when(j + 2 < iters)
            def _():
                start(j + 2, 0)

            finish(j + 1, 1)
            return carry

        lax.fori_loop(0, iters // 2, body, 0)
        if iters % 2 == 1:
            finish(iters - 1, 0)
        plsc.subcore_barrier()
        pltpu.sync_copy(aggr.at[pl.ds(sid * rpt, rpt)], big_v)
        pltpu.sync_copy(big_v, part.at[cid, pl.ds(sid * rpt, rpt)])

    return scatter_k


def _h0_body(xb, nab, W, b, out):
    xa = jnp.concatenate([xb[...], nab[...]], axis=1)
    out[...] = jnp.dot(xa, W[...], preferred_element_type=F32) + b[...]


def _msg_body(gd, gs, eattr, eW, eb, Wfull, bias, out, *, ND):
    ea = jnp.dot(eattr[...], eW[...], preferred_element_type=F32) + eb[...]
    z = jnp.concatenate([gd[...], gs[...], ea], axis=1)
    pre = jnp.dot(z, Wfull[...], preferred_element_type=F32) + bias[...]
    a = pre[:, :ND]
    b = pre[:, ND:]
    out[...] = jax.nn.sigmoid(a) * jax.nn.softplus(b)


def _bn_body(pa, pb, h, g, be, out, s_acc, ss_acc, *, inv_n):
    ph = pl.program_id(0)
    i = pl.program_id(1)

    @pl.when(jnp.logical_and(ph == 0, i == 0))
    def _():
        s_acc[...] = jnp.zeros_like(s_acc)
        ss_acc[...] = jnp.zeros_like(ss_acc)

    aggr = pa[0] + pb[0]

    @pl.when(ph == 0)
    def _():
        s_acc[...] += jnp.sum(aggr, axis=0, keepdims=True)

    @pl.when(ph == 1)
    def _():
        d = aggr - s_acc[...] * inv_n
        ss_acc[...] += jnp.sum(d * d, axis=0, keepdims=True)

    @pl.when(ph == 2)
    def _():
        m = s_acc[...] * inv_n
        v = ss_acc[...] * inv_n
        scale = g[...] * lax.rsqrt(v + EPS)
        out[...] = h[...] + (aggr - m) * scale + be[...]


def _stage_body(h, W, b, g, be, out, y_buf, s_acc, ss_acc, *, use_mm, Bn,
                inv_n):
    ph = pl.program_id(0)
    i = pl.program_id(1)

    @pl.when(jnp.logical_and(ph == 0, i == 0))
    def _():
        s_acc[...] = jnp.zeros_like(s_acc)
        ss_acc[...] = jnp.zeros_like(ss_acc)

    @pl.when(ph == 0)
    def _():
        if use_mm:
            y = jnp.dot(h[...], W[...], preferred_element_type=F32) + b[...]
        else:
            y = h[...]
        y_buf[pl.ds(i * Bn, Bn), :] = y
        s_acc[...] += jnp.sum(y, axis=0, keepdims=True)

    @pl.when(ph == 1)
    def _():
        d = y_buf[pl.ds(i * Bn, Bn), :] - s_acc[...] * inv_n
        ss_acc[...] += jnp.sum(d * d, axis=0, keepdims=True)

    @pl.when(ph == 2)
    def _():
        y = y_buf[pl.ds(i * Bn, Bn), :]
        m = s_acc[...] * inv_n
        v = ss_acc[...] * inv_n
        out[...] = jax.nn.softplus(
            g[...] * (y - m) * lax.rsqrt(v + EPS) + be[...])


def _final_body(h, W, b, out):
    out[...] = jnp.dot(h[...], W[...], preferred_element_type=F32) + b[...]


def _full(shape):
    return pl.BlockSpec(shape, lambda *args: (0,) * len(shape))


def kernel(x, node_attr, edge_attr, edge_index, node_W, node_b, edge_W,
           edge_b, conv_Wf, conv_bf, conv_Ws, conv_bs, conv_g, conv_be,
           sbn_g, sbn_b, fc_W, fc_b, fbn_g, fbn_b, head_W, head_b, head_g,
           head_be, out_W, out_b):
    N = x.shape[0]
    E = edge_attr.shape[0]
    ND = node_W.shape[1]
    ED = edge_W.shape[1]
    NL = conv_Wf.shape[0]
    NFC = head_W.shape[0]
    H = fc_W.shape[1]
    Bn = 1000
    Be = 2000
    K = 80
    nb = N // Bn
    inv_n = 1.0 / N

    src = edge_index[0]
    dst = edge_index[1]

    h = pl.pallas_call(
        _h0_body,
        grid=(nb,),
        in_specs=[
            pl.BlockSpec((Bn, 1), lambda i: (i, 0)),
            pl.BlockSpec((Bn, 2), lambda i: (i, 0)),
            _full((3, ND)),
            _full((1, ND)),
        ],
        out_specs=pl.BlockSpec((Bn, ND), lambda i: (i, 0)),
        out_shape=jax.ShapeDtypeStruct((N, ND), F32),
    )(x.reshape(N, 1), node_attr, node_W, node_b.reshape(1, ND))

    half = N // 2
    Hp = (half // (NS * 8) + 1) * (NS * 8)
    bpc = half // Bn
    NW = NC * NS
    w = E // NW
    wA = ((w // 2 + 127) // 128) * 128
    neA = wA * NW
    chunks = [(0, neA), (neA, E - neA)]

    def _blk(n_, cap):
        for k in range(cap, 7, -8):
            if n_ % k == 0:
                return k
        raise ValueError(n_)

    gathers = [_make_gather(N, E, ND, _blk(n_ // NW, 128), e0, n_)
               for e0, n_ in chunks]
    scatters = [_make_scatter(half, Hp, E, ND, K, e0, n_) for e0, n_ in chunks]
    zeros_init = jnp.zeros((Hp // NS, ND), F32)

    ZD = 2 * ND + ED

    def make_msg_call(e0, n_):
        Bee = _blk(n_, 2560)
        assert e0 % Bee == 0
        blk0 = e0 // Bee
        return pl.pallas_call(
            functools.partial(_msg_body, ND=ND),
            grid=(n_ // Bee,),
            in_specs=[
                pl.BlockSpec((Bee, ND), lambda i: (i, 0)),
                pl.BlockSpec((Bee, ND), lambda i: (i, 0)),
                pl.BlockSpec((Bee, 2), lambda i: (i + blk0, 0)),
                _full((2, ED)),
                _full((1, ED)),
                _full((ZD, 2 * ND)),
                _full((1, 2 * ND)),
            ],
            out_specs=pl.BlockSpec((Bee, ND), lambda i: (i, 0)),
            out_shape=jax.ShapeDtypeStruct((n_, ND), F32),
        )

    msg_calls = [make_msg_call(e0, n_) for e0, n_ in chunks]
    eb2 = edge_b.reshape(1, ED)
    bias_all = jnp.concatenate([conv_bf, conv_bs], axis=1).reshape(NL, 1,
                                                                   2 * ND)
    Wfull_all = jnp.concatenate([conv_Wf, conv_Ws], axis=2)

    part_spec = pl.BlockSpec((1, Bn, ND), lambda p, i: (i // bpc, i % bpc, 0))
    bn_call = pl.pallas_call(
        functools.partial(_bn_body, inv_n=inv_n),
        grid=(3, nb),
        in_specs=[
            part_spec,
            part_spec,
            pl.BlockSpec((Bn, ND), lambda p, i: (i, 0)),
            _full((1, ND)),
            _full((1, ND)),
        ],
        out_specs=pl.BlockSpec((Bn, ND), lambda p, i: (i, 0)),
        out_shape=jax.ShapeDtypeStruct((N, ND), F32),
        scratch_shapes=[pltpu.VMEM((1, ND), F32), pltpu.VMEM((1, ND), F32)],
    )

    for l in range(NL):
        gs_pairs = [g_k(h, dst, src) for g_k in gathers]
        msgs = [m_c(gd, gs, edge_attr, edge_W, eb2, Wfull_all[l], bias_all[l])
                for m_c, (gd, gs) in zip(msg_calls, gs_pairs)]
        parts = [s_k(m, dst, zeros_init) for s_k, m in zip(scatters, msgs)]
        h = bn_call(parts[0], parts[1], h, conv_g[l].reshape(1, ND),
                    conv_be[l].reshape(1, ND))

    def stage(h, W, b, g, be, use_mm, Dout):
        return pl.pallas_call(
            functools.partial(_stage_body, use_mm=use_mm, Bn=Bn, inv_n=inv_n),
            grid=(3, nb),
            in_specs=[
                pl.BlockSpec((Bn, ND), lambda p, i: (i, 0)),
                _full((ND, Dout)),
                _full((1, Dout)),
                _full((1, Dout)),
                _full((1, Dout)),
            ],
            out_specs=pl.BlockSpec((Bn, Dout), lambda p, i: (i, 0)),
            out_shape=jax.ShapeDtypeStruct((N, Dout), F32),
            scratch_shapes=[
                pltpu.VMEM((N, Dout), F32),
                pltpu.VMEM((1, Dout), F32),
                pltpu.VMEM((1, Dout), F32),
            ],
        )(h, W, b.reshape(1, Dout), g.reshape(1, Dout), be.reshape(1, Dout))

    eye = jnp.eye(ND, dtype=F32)
    h = stage(h, eye, jnp.zeros((ND,), F32), sbn_g, sbn_b, False, ND)
    h = stage(h, fc_W, fc_b, fbn_g, fbn_b, True, H)
    for i in range(NFC):
        h = stage(h, head_W[i], head_b[i], head_g[i], head_be[i], True, H)

    out = pl.pallas_call(
        _final_body,
        grid=(nb,),
        in_specs=[
            pl.BlockSpec((Bn, H), lambda i: (i, 0)),
            _full((H, 1)),
            _full((1, 1)),
        ],
        out_specs=pl.BlockSpec((Bn, 1), lambda i: (i, 0)),
        out_shape=jax.ShapeDtypeStruct((N, 1), F32),
    )(h, out_W, out_b.reshape(1, 1))
    return out


# speedup vs baseline: 2.9877x; 1.0108x over previous
import functools

import jax
import jax.numpy as jnp
from jax import lax
from jax.experimental import pallas as pl
from jax.experimental.pallas import tpu as pltpu
from jax.experimental.pallas import tpu_sc as plsc

EPS = 1e-5
NC = 2
NS = 16
F32 = jnp.float32


@functools.lru_cache(maxsize=None)
def _make_gather(N, E, D, K, e0, ne):
    NW = NC * NS
    chunk = ne // NW
    iters = chunk // K
    assert chunk % K == 0 and ne % NW == 0 and iters >= 3 and e0 % 8 == 0
    mesh = plsc.VectorSubcoreMesh(core_axis_name="c", subcore_axis_name="s")

    @functools.partial(
        pl.kernel,
        out_type=(jax.ShapeDtypeStruct((ne, D), F32),
                  jax.ShapeDtypeStruct((ne, D), F32)),
        mesh=mesh,
        scratch_types=[
            pltpu.VMEM((chunk,), jnp.int32),
            pltpu.VMEM((chunk,), jnp.int32),
            [pltpu.VMEM((K, D), F32)] * 4,
            [pltpu.SemaphoreType.DMA] * 4,
        ],
    )
    def gather_k(table, dst_i, src_i, gd, gs, idx_d, idx_s, rows, sems):
        wid = lax.axis_index("s") * NC + lax.axis_index("c")
        base0 = wid * chunk
        pltpu.sync_copy(dst_i.at[pl.ds(e0 + base0, chunk)], idx_d)
        pltpu.sync_copy(src_i.at[pl.ds(e0 + base0, chunk)], idx_s)

        def start(j, b):
            off = j * K
            pltpu.async_copy(table.at[idx_d.at[pl.ds(off, K)]],
                             rows[2 * b], sems[2 * b])
            pltpu.async_copy(table.at[idx_s.at[pl.ds(off, K)]],
                             rows[2 * b + 1], sems[2 * b + 1])

        def finish(j, b):
            base = base0 + j * K
            pltpu.make_async_copy(table.at[idx_d.at[pl.ds(0, K)]],
                                  rows[2 * b], sems[2 * b]).wait()
            pltpu.sync_copy(rows[2 * b], gd.at[pl.ds(base, K)])
            pltpu.make_async_copy(table.at[idx_s.at[pl.ds(0, K)]],
                                  rows[2 * b + 1], sems[2 * b + 1]).wait()
            pltpu.sync_copy(rows[2 * b + 1], gs.at[pl.ds(base, K)])

        start(0, 0)

        def body(t, carry):
            j = 2 * t
            start(j + 1, 1)
            finish(j, 0)

            @pl.when(j + 2 < iters)
            def _():
                start(j + 2, 0)

            finish(j + 1, 1)
            return carry

        lax.fori_loop(0, iters // 2, body, 0)
        if iters % 2 == 1:
            finish(iters - 1, 0)

    return gather_k


@functools.lru_cache(maxsize=None)
def _make_scatter(half, Hp, E, D, K, e0, ne):
    chunk = ne // NS
    iters = chunk // K
    rpt = Hp // NS
    assert chunk % K == 0 and K % 16 == 0 and rpt % 8 == 0 and Hp > half
    assert iters >= 3 and e0 % 8 == 0
    mesh = plsc.VectorSubcoreMesh(core_axis_name="c", subcore_axis_name="s")

    @functools.partial(
        pl.kernel,
        out_type=jax.ShapeDtypeStruct((NC, Hp, D), F32),
        mesh=mesh,
        scratch_types=[
            pltpu.VMEM((chunk,), jnp.int32),
            [pltpu.VMEM((K,), jnp.int32)] * 2,
            [pltpu.VMEM((K, D), F32)] * 2,
            [pltpu.SemaphoreType.DMA] * 2,
            pltpu.VMEM((rpt, D), F32),
            pltpu.VMEM_SHARED((Hp, D), F32),
        ],
    )
    def scatter_k(msg, dst_i, zeros, part, idx_all, idx2, rows, sems, big_v,
                  aggr):
        cid = lax.axis_index("c")
        sid = lax.axis_index("s")
        lo = cid * half
        base0 = sid * chunk
        pltpu.sync_copy(zeros, big_v)
        pltpu.sync_copy(big_v, aggr.at[pl.ds(sid * rpt, rpt)])
        pltpu.sync_copy(dst_i.at[pl.ds(e0 + base0, chunk)], idx_all)
        plsc.subcore_barrier()

        def start(j, b):
            pltpu.async_copy(msg.at[pl.ds(base0 + j * K, K)], rows[b],
                             sems[b])

        def finish(j, b):
            off = j * K
            for t in range(K // 16):
                v = idx_all[pl.ds(off + t * 16, 16)] - lo
                ok = jnp.logical_and(v >= 0, v < half)
                idx2[b][pl.ds(t * 16, 16)] = jnp.where(ok, v, half)
            pltpu.make_async_copy(msg.at[pl.ds(0, K)], rows[b],
                                  sems[b]).wait()
            pltpu.sync_copy(rows[b], aggr.at[idx2[b]], add=True)

        start(0, 0)

        def body(t, carry):
            j = 2 * t
            start(j + 1, 1)
            finish(j, 0)

            @pl.when(j + 2 < iters)
            def _():
                start(j + 2, 0)

            finish(j + 1, 1)
            return carry

        lax.fori_loop(0, iters // 2, body, 0)
        if iters % 2 == 1:
            finish(iters - 1, 0)
        plsc.subcore_barrier()
        pltpu.sync_copy(aggr.at[pl.ds(sid * rpt, rpt)], big_v)
        pltpu.sync_copy(big_v, part.at[cid, pl.ds(sid * rpt, rpt)])

    return scatter_k


def _h0_body(xb, nab, W, b, out):
    xa = jnp.concatenate([xb[...], nab[...]], axis=1)
    out[...] = jnp.dot(xa, W[...], preferred_element_type=F32) + b[...]


def _ea_body(ea2, eW, eb, out):
    out[...] = lax.dot_general(ea2[...], eW[...], (((0,), (0,)), ((), ())),
                               preferred_element_type=F32) + eb[...]


def _msg_body(gd, gs, ea16, Wfull, bias, out, *, ND):
    z = jnp.concatenate([gd[...], gs[...], ea16[...]], axis=1)
    pre = jnp.dot(z, Wfull[...], preferred_element_type=F32) + bias[...]
    a = pre[:, :ND]
    b = pre[:, ND:]
    out[...] = jax.nn.sigmoid(a) * jax.nn.softplus(b)


def _bn_body(pa, pb, h, g, be, out, s_acc, ss_acc, *, inv_n):
    ph = pl.program_id(0)
    i = pl.program_id(1)

    @pl.when(jnp.logical_and(ph == 0, i == 0))
    def _():
        s_acc[...] = jnp.zeros_like(s_acc)
        ss_acc[...] = jnp.zeros_like(ss_acc)

    aggr = pa[0] + pb[0]

    @pl.when(ph == 0)
    def _():
        s_acc[...] += jnp.sum(aggr, axis=0, keepdims=True)

    @pl.when(ph == 1)
    def _():
        d = aggr - s_acc[...] * inv_n
        ss_acc[...] += jnp.sum(d * d, axis=0, keepdims=True)

    @pl.when(ph == 2)
    def _():
        m = s_acc[...] * inv_n
        v = ss_acc[...] * inv_n
        scale = g[...] * lax.rsqrt(v + EPS)
        out[...] = h[...] + (aggr - m) * scale + be[...]


def _stage_body(h, W, b, g, be, out, y_buf, s_acc, ss_acc, *, use_mm, Bn,
                inv_n):
    ph = pl.program_id(0)
    i = pl.program_id(1)

    @pl.when(jnp.logical_and(ph == 0, i == 0))
    def _():
        s_acc[...] = jnp.zeros_like(s_acc)
        ss_acc[...] = jnp.zeros_like(ss_acc)

    @pl.when(ph == 0)
    def _():
        if use_mm:
            y = jnp.dot(h[...], W[...], preferred_element_type=F32) + b[...]
        else:
            y = h[...]
        y_buf[pl.ds(i * Bn, Bn), :] = y
        s_acc[...] += jnp.sum(y, axis=0, keepdims=True)

    @pl.when(ph == 1)
    def _():
        d = y_buf[pl.ds(i * Bn, Bn), :] - s_acc[...] * inv_n
        ss_acc[...] += jnp.sum(d * d, axis=0, keepdims=True)

    @pl.when(ph == 2)
    def _():
        y = y_buf[pl.ds(i * Bn, Bn), :]
        m = s_acc[...] * inv_n
        v = ss_acc[...] * inv_n
        out[...] = jax.nn.softplus(
            g[...] * (y - m) * lax.rsqrt(v + EPS) + be[...])


def _tail_body(h, W_all, b_all, g_all, be_all, oW, ob, out, hcur, y_buf,
               s_acc, ss_acc, *, Bn, inv_n, nstage):
    st = pl.program_id(0)
    ph = pl.program_id(1)
    i = pl.program_id(2)
    ds = pl.ds(i * Bn, Bn)

    @pl.when(jnp.logical_and(ph == 0, i == 0))
    def _():
        s_acc[...] = jnp.zeros_like(s_acc)
        ss_acc[...] = jnp.zeros_like(ss_acc)

    @pl.when(ph == 0)
    def _():
        xsel = jnp.where(st == 0, h[...], hcur[ds, :])
        y = jnp.dot(xsel, W_all[0], preferred_element_type=F32) + b_all[0]
        y_buf[ds, :] = y
        s_acc[...] += jnp.sum(y, axis=0, keepdims=True)

    @pl.when(ph == 1)
    def _():
        d = y_buf[ds, :] - s_acc[...] * inv_n
        ss_acc[...] += jnp.sum(d * d, axis=0, keepdims=True)

    @pl.when(ph == 2)
    def _():
        y = y_buf[ds, :]
        m = s_acc[...] * inv_n
        v = ss_acc[...] * inv_n
        z = jax.nn.softplus(
            g_all[0] * (y - m) * lax.rsqrt(v + EPS) + be_all[0])

        @pl.when(st < nstage - 1)
        def _():
            hcur[ds, :] = z

        @pl.when(st == nstage - 1)
        def _():
            out[...] = jnp.dot(z, oW[...], preferred_element_type=F32) + ob[...]


def _full(shape):
    return pl.BlockSpec(shape, lambda *args: (0,) * len(shape))


def kernel(x, node_attr, edge_attr, edge_index, node_W, node_b, edge_W,
           edge_b, conv_Wf, conv_bf, conv_Ws, conv_bs, conv_g, conv_be,
           sbn_g, sbn_b, fc_W, fc_b, fbn_g, fbn_b, head_W, head_b, head_g,
           head_be, out_W, out_b):
    N = x.shape[0]
    E = edge_attr.shape[0]
    ND = node_W.shape[1]
    ED = edge_W.shape[1]
    NL = conv_Wf.shape[0]
    NFC = head_W.shape[0]
    H = fc_W.shape[1]
    Bn = 1000
    Be = 2000
    K = 80
    nb = N // Bn
    inv_n = 1.0 / N

    src = edge_index[0]
    dst = edge_index[1]

    h = pl.pallas_call(
        _h0_body,
        grid=(nb,),
        in_specs=[
            pl.BlockSpec((Bn, 1), lambda i: (i, 0)),
            pl.BlockSpec((Bn, 2), lambda i: (i, 0)),
            _full((3, ND)),
            _full((1, ND)),
        ],
        out_specs=pl.BlockSpec((Bn, ND), lambda i: (i, 0)),
        out_shape=jax.ShapeDtypeStruct((N, ND), F32),
    )(x.reshape(N, 1), node_attr, node_W, node_b.reshape(1, ND))

    half = N // 2
    Hp = (half // (NS * 8) + 1) * (NS * 8)
    bpc = half // Bn
    NW = NC * NS
    w = E // NW
    wA = ((w // 2 + 127) // 128) * 128
    neA = wA * NW
    chunks = [(0, neA), (neA, E - neA)]

    def _blk(n_, cap):
        for k in range(cap, 7, -8):
            if n_ % k == 0:
                return k
        raise ValueError(n_)

    gathers = [_make_gather(N, E, ND, _blk(n_ // NW, 128), e0, n_)
               for e0, n_ in chunks]
    scatters = [_make_scatter(half, Hp, E, ND, K, e0, n_) for e0, n_ in chunks]
    zeros_init = jnp.zeros((Hp // NS, ND), F32)

    ZD = 2 * ND + ED

    def make_msg_call(e0, n_):
        for Bee in range(4096, 7, -8):
            if n_ % Bee == 0 and e0 % Bee == 0:
                break
        blk0 = e0 // Bee
        return pl.pallas_call(
            functools.partial(_msg_body, ND=ND),
            grid=(n_ // Bee,),
            in_specs=[
                pl.BlockSpec((Bee, ND), lambda i: (i, 0)),
                pl.BlockSpec((Bee, ND), lambda i: (i, 0)),
                pl.BlockSpec((Bee, ED), lambda i: (i + blk0, 0)),
                _full((ZD, 2 * ND)),
                _full((1, 2 * ND)),
            ],
            out_specs=pl.BlockSpec((Bee, ND), lambda i: (i, 0)),
            out_shape=jax.ShapeDtypeStruct((n_, ND), F32),
        )

    msg_calls = [make_msg_call(e0, n_) for e0, n_ in chunks]
    Bea = 2560
    ea16 = pl.pallas_call(
        _ea_body,
        grid=(E // Bea,),
        in_specs=[
            pl.BlockSpec((2, Bea), lambda i: (0, i)),
            _full((2, ED)),
            _full((1, ED)),
        ],
        out_specs=pl.BlockSpec((Bea, ED), lambda i: (i, 0)),
        out_shape=jax.ShapeDtypeStruct((E, ED), F32),
    )(edge_attr.T, edge_W, edge_b.reshape(1, ED))
    bias_all = jnp.concatenate([conv_bf, conv_bs], axis=1).reshape(NL, 1,
                                                                   2 * ND)
    Wfull_all = jnp.concatenate([conv_Wf, conv_Ws], axis=2)

    part_spec = pl.BlockSpec((1, Bn, ND), lambda p, i: (i // bpc, i % bpc, 0))
    bn_call = pl.pallas_call(
        functools.partial(_bn_body, inv_n=inv_n),
        grid=(3, nb),
        in_specs=[
            part_spec,
            part_spec,
            pl.BlockSpec((Bn, ND), lambda p, i: (i, 0)),
            _full((1, ND)),
            _full((1, ND)),
        ],
        out_specs=pl.BlockSpec((Bn, ND), lambda p, i: (i, 0)),
        out_shape=jax.ShapeDtypeStruct((N, ND), F32),
        scratch_shapes=[pltpu.VMEM((1, ND), F32), pltpu.VMEM((1, ND), F32)],
    )

    for l in range(NL):
        gs_pairs = [g_k(h, dst, src) for g_k in gathers]
        msgs = [m_c(gd, gs, ea16, Wfull_all[l], bias_all[l])
                for m_c, (gd, gs) in zip(msg_calls, gs_pairs)]
        parts = [s_k(m, dst, zeros_init) for s_k, m in zip(scatters, msgs)]
        h = bn_call(parts[0], parts[1], h, conv_g[l].reshape(1, ND),
                    conv_be[l].reshape(1, ND))

    def stage(h, W, b, g, be, use_mm, Dout):
        return pl.pallas_call(
            functools.partial(_stage_body, use_mm=use_mm, Bn=Bn, inv_n=inv_n),
            grid=(3, nb),
            in_specs=[
                pl.BlockSpec((Bn, ND), lambda p, i: (i, 0)),
                _full((ND, Dout)),
                _full((1, Dout)),
                _full((1, Dout)),
                _full((1, Dout)),
            ],
            out_specs=pl.BlockSpec((Bn, Dout), lambda p, i: (i, 0)),
            out_shape=jax.ShapeDtypeStruct((N, Dout), F32),
            scratch_shapes=[
                pltpu.VMEM((N, Dout), F32),
                pltpu.VMEM((1, Dout), F32),
                pltpu.VMEM((1, Dout), F32),
            ],
        )(h, W, b.reshape(1, Dout), g.reshape(1, Dout), be.reshape(1, Dout))

    eye = jnp.eye(ND, dtype=F32)
    h = stage(h, eye, jnp.zeros((ND,), F32), sbn_g, sbn_b, False, ND)

    nstage = NFC + 1
    W_all = jnp.concatenate([fc_W.reshape(1, ND, H), head_W], axis=0)
    b_all = jnp.concatenate([fc_b.reshape(1, H), head_b], axis=0)
    g_all = jnp.concatenate([fbn_g.reshape(1, H), head_g], axis=0)
    be_all = jnp.concatenate([fbn_b.reshape(1, H), head_be], axis=0)
    svec = pl.BlockSpec((1, 1, H), lambda st, p, i: (st, 0, 0))
    out = pl.pallas_call(
        functools.partial(_tail_body, Bn=Bn, inv_n=inv_n, nstage=nstage),
        grid=(nstage, 3, nb),
        in_specs=[
            pl.BlockSpec((Bn, ND), lambda st, p, i: (i, 0)),
            pl.BlockSpec((1, ND, H), lambda st, p, i: (st, 0, 0)),
            svec,
            svec,
            svec,
            _full((H, 1)),
            _full((1, 1)),
        ],
        out_specs=pl.BlockSpec((Bn, 1), lambda st, p, i: (i, 0)),
        out_shape=jax.ShapeDtypeStruct((N, 1), F32),
        scratch_shapes=[
            pltpu.VMEM((N, H), F32),
            pltpu.VMEM((N, H), F32),
            pltpu.VMEM((1, H), F32),
            pltpu.VMEM((1, H), F32),
        ],
    )(h, W_all, b_all.reshape(nstage, 1, H), g_all.reshape(nstage, 1, H),
      be_all.reshape(nstage, 1, H), out_W, out_b.reshape(1, 1))
    return out


# speedup vs baseline: 3.0878x; 1.0335x over previous
import functools

import jax
import jax.numpy as jnp
from jax import lax
from jax.experimental import pallas as pl
from jax.experimental.pallas import tpu as pltpu
from jax.experimental.pallas import tpu_sc as plsc

EPS = 1e-5
NC = 2
NS = 16
F32 = jnp.float32


@functools.lru_cache(maxsize=None)
def _make_gather(N, E, D, K, e0, ne):
    NW = NC * NS
    chunk = ne // NW
    iters = chunk // K
    assert chunk % K == 0 and ne % NW == 0 and iters >= 3 and e0 % 8 == 0
    mesh = plsc.VectorSubcoreMesh(core_axis_name="c", subcore_axis_name="s")

    @functools.partial(
        pl.kernel,
        out_type=(jax.ShapeDtypeStruct((ne, D), F32),
                  jax.ShapeDtypeStruct((ne, D), F32)),
        mesh=mesh,
        scratch_types=[
            pltpu.VMEM((chunk,), jnp.int32),
            pltpu.VMEM((chunk,), jnp.int32),
            [pltpu.VMEM((K, D), F32)] * 4,
            [pltpu.SemaphoreType.DMA] * 4,
        ],
    )
    def gather_k(table, dst_i, src_i, gd, gs, idx_d, idx_s, rows, sems):
        wid = lax.axis_index("s") * NC + lax.axis_index("c")
        base0 = wid * chunk
        pltpu.sync_copy(dst_i.at[pl.ds(e0 + base0, chunk)], idx_d)
        pltpu.sync_copy(src_i.at[pl.ds(e0 + base0, chunk)], idx_s)

        def start(j, b):
            off = j * K
            pltpu.async_copy(table.at[idx_d.at[pl.ds(off, K)]],
                             rows[2 * b], sems[2 * b])
            pltpu.async_copy(table.at[idx_s.at[pl.ds(off, K)]],
                             rows[2 * b + 1], sems[2 * b + 1])

        def finish(j, b):
            base = base0 + j * K
            pltpu.make_async_copy(table.at[idx_d.at[pl.ds(0, K)]],
                                  rows[2 * b], sems[2 * b]).wait()
            pltpu.sync_copy(rows[2 * b], gd.at[pl.ds(base, K)])
            pltpu.make_async_copy(table.at[idx_s.at[pl.ds(0, K)]],
                                  rows[2 * b + 1], sems[2 * b + 1]).wait()
            pltpu.sync_copy(rows[2 * b + 1], gs.at[pl.ds(base, K)])

        start(0, 0)

        def body(t, carry):
            j = 2 * t
            start(j + 1, 1)
            finish(j, 0)

            @pl.when(j + 2 < iters)
            def _():
                start(j + 2, 0)

            finish(j + 1, 1)
            return carry

        lax.fori_loop(0, iters // 2, body, 0)
        if iters % 2 == 1:
            finish(iters - 1, 0)

    return gather_k


@functools.lru_cache(maxsize=None)
def _make_scatter(half, Hp, E, D, K, e0, ne):
    chunk = ne // NS
    iters = chunk // K
    rpt = Hp // NS
    assert chunk % K == 0 and K % 16 == 0 and rpt % 8 == 0 and Hp > half
    assert iters >= 3 and e0 % 8 == 0
    mesh = plsc.VectorSubcoreMesh(core_axis_name="c", subcore_axis_name="s")

    @functools.partial(
        pl.kernel,
        out_type=jax.ShapeDtypeStruct((NC, Hp, D), F32),
        mesh=mesh,
        scratch_types=[
            pltpu.VMEM((chunk,), jnp.int32),
            [pltpu.VMEM((K,), jnp.int32)] * 2,
            [pltpu.VMEM((K, D), F32)] * 2,
            [pltpu.SemaphoreType.DMA] * 2,
            pltpu.VMEM((rpt, D), F32),
            pltpu.VMEM_SHARED((Hp, D), F32),
        ],
    )
    def scatter_k(msg, dst_i, zeros, part, idx_all, idx2, rows, sems, big_v,
                  aggr):
        cid = lax.axis_index("c")
        sid = lax.axis_index("s")
        lo = cid * half
        base0 = sid * chunk
        pltpu.sync_copy(zeros, big_v)
        pltpu.sync_copy(big_v, aggr.at[pl.ds(sid * rpt, rpt)])
        pltpu.sync_copy(dst_i.at[pl.ds(e0 + base0, chunk)], idx_all)
        plsc.subcore_barrier()

        def start(j, b):
            pltpu.async_copy(msg.at[pl.ds(base0 + j * K, K)], rows[b],
                             sems[b])

        def finish(j, b):
            off = j * K
            for t in range(K // 16):
                v = idx_all[pl.ds(off + t * 16, 16)] - lo
                ok = jnp.logical_and(v >= 0, v < half)
                idx2[b][pl.ds(t * 16, 16)] = jnp.where(ok, v, half)
            pltpu.make_async_copy(msg.at[pl.ds(0, K)], rows[b],
                                  sems[b]).wait()
            pltpu.sync_copy(rows[b], aggr.at[idx2[b]], add=True)

        start(0, 0)

        def body(t, carry):
            j = 2 * t
            start(j + 1, 1)
            finish(j, 0)

            @pl.when(j + 2 < iters)
            def _():
                start(j + 2, 0)

            finish(j + 1, 1)
            return carry

        lax.fori_loop(0, iters // 2, body, 0)
        if iters % 2 == 1:
            finish(iters - 1, 0)
        plsc.subcore_barrier()
        pltpu.sync_copy(aggr.at[pl.ds(sid * rpt, rpt)], big_v)
        pltpu.sync_copy(big_v, part.at[cid, pl.ds(sid * rpt, rpt)])

    return scatter_k


def _h0_body(xb, nab, W, b, out):
    xa = jnp.concatenate([xb[...], nab[...]], axis=1)
    out[...] = jnp.dot(xa, W[...], preferred_element_type=F32) + b[...]


def _ea_body(ea2, eW, eb, out):
    out[...] = lax.dot_general(ea2[...], eW[...], (((0,), (0,)), ((), ())),
                               preferred_element_type=F32) + eb[...]


def _msg_body(gd, gs, ea16, Wfull, bias, out, *, ND):
    z = jnp.concatenate([gd[...], gs[...], ea16[...]], axis=1)
    pre = jnp.dot(z, Wfull[...], preferred_element_type=F32) + bias[...]
    a = pre[:, :ND]
    b = pre[:, ND:]
    out[...] = jax.nn.sigmoid(a) * jax.nn.softplus(b)


def _bn_body(pa, pb, h, g, be, out, s_acc, ss_acc, *, inv_n):
    ph = pl.program_id(0)
    i = pl.program_id(1)

    @pl.when(jnp.logical_and(ph == 0, i == 0))
    def _():
        s_acc[...] = jnp.zeros_like(s_acc)
        ss_acc[...] = jnp.zeros_like(ss_acc)

    aggr = pa[0] + pb[0]

    @pl.when(ph == 0)
    def _():
        s_acc[...] += jnp.sum(aggr, axis=0, keepdims=True)

    @pl.when(ph == 1)
    def _():
        d = aggr - s_acc[...] * inv_n
        ss_acc[...] += jnp.sum(d * d, axis=0, keepdims=True)

    @pl.when(ph == 2)
    def _():
        m = s_acc[...] * inv_n
        v = ss_acc[...] * inv_n
        scale = g[...] * lax.rsqrt(v + EPS)
        out[...] = h[...] + (aggr - m) * scale + be[...]


def _stage_body(h, W, b, g, be, out, y_buf, s_acc, ss_acc, *, use_mm, Bn,
                inv_n):
    ph = pl.program_id(0)
    i = pl.program_id(1)

    @pl.when(jnp.logical_and(ph == 0, i == 0))
    def _():
        s_acc[...] = jnp.zeros_like(s_acc)
        ss_acc[...] = jnp.zeros_like(ss_acc)

    @pl.when(ph == 0)
    def _():
        if use_mm:
            y = jnp.dot(h[...], W[...], preferred_element_type=F32) + b[...]
        else:
            y = h[...]
        y_buf[pl.ds(i * Bn, Bn), :] = y
        s_acc[...] += jnp.sum(y, axis=0, keepdims=True)

    @pl.when(ph == 1)
    def _():
        d = y_buf[pl.ds(i * Bn, Bn), :] - s_acc[...] * inv_n
        ss_acc[...] += jnp.sum(d * d, axis=0, keepdims=True)

    @pl.when(ph == 2)
    def _():
        y = y_buf[pl.ds(i * Bn, Bn), :]
        m = s_acc[...] * inv_n
        v = ss_acc[...] * inv_n
        out[...] = jax.nn.softplus(
            g[...] * (y - m) * lax.rsqrt(v + EPS) + be[...])


def _tail_body(h, W_all, b_all, g_all, be_all, oW, ob, out, hcur, y_buf,
               s_acc, ss_acc, *, Bn, inv_n, nstage):
    st = pl.program_id(0)
    ph = pl.program_id(1)
    i = pl.program_id(2)
    ds = pl.ds(i * Bn, Bn)

    @pl.when(jnp.logical_and(ph == 0, i == 0))
    def _():
        s_acc[...] = jnp.zeros_like(s_acc)
        ss_acc[...] = jnp.zeros_like(ss_acc)

    @pl.when(ph == 0)
    def _():
        xsel = jnp.where(st == 0, h[...], hcur[ds, :])
        y = jnp.dot(xsel, W_all[0], preferred_element_type=F32) + b_all[0]
        y_buf[ds, :] = y
        s_acc[...] += jnp.sum(y, axis=0, keepdims=True)

    @pl.when(ph == 1)
    def _():
        d = y_buf[ds, :] - s_acc[...] * inv_n
        ss_acc[...] += jnp.sum(d * d, axis=0, keepdims=True)

    @pl.when(ph == 2)
    def _():
        y = y_buf[ds, :]
        m = s_acc[...] * inv_n
        v = ss_acc[...] * inv_n
        z = jax.nn.softplus(
            g_all[0] * (y - m) * lax.rsqrt(v + EPS) + be_all[0])

        @pl.when(st < nstage - 1)
        def _():
            hcur[ds, :] = z

        @pl.when(st == nstage - 1)
        def _():
            out[...] = jnp.dot(z, oW[...], preferred_element_type=F32) + ob[...]


def _full(shape):
    return pl.BlockSpec(shape, lambda *args: (0,) * len(shape))


def kernel(x, node_attr, edge_attr, edge_index, node_W, node_b, edge_W,
           edge_b, conv_Wf, conv_bf, conv_Ws, conv_bs, conv_g, conv_be,
           sbn_g, sbn_b, fc_W, fc_b, fbn_g, fbn_b, head_W, head_b, head_g,
           head_be, out_W, out_b):
    N = x.shape[0]
    E = edge_attr.shape[0]
    ND = node_W.shape[1]
    ED = edge_W.shape[1]
    NL = conv_Wf.shape[0]
    NFC = head_W.shape[0]
    H = fc_W.shape[1]
    Bn = 1000
    Bt = 5000
    K = 80
    nb = N // Bn
    ntb = N // Bt
    inv_n = 1.0 / N

    src = edge_index[0]
    dst = edge_index[1]

    h = pl.pallas_call(
        _h0_body,
        grid=(nb,),
        in_specs=[
            pl.BlockSpec((Bn, 1), lambda i: (i, 0)),
            pl.BlockSpec((Bn, 2), lambda i: (i, 0)),
            _full((3, ND)),
            _full((1, ND)),
        ],
        out_specs=pl.BlockSpec((Bn, ND), lambda i: (i, 0)),
        out_shape=jax.ShapeDtypeStruct((N, ND), F32),
    )(x.reshape(N, 1), node_attr, node_W, node_b.reshape(1, ND))

    half = N // 2
    Hp = (half // (NS * 8) + 1) * (NS * 8)
    bpc = half // Bn
    NW = NC * NS
    w = E // NW
    wA = ((w // 2 + 127) // 128) * 128
    neA = wA * NW
    chunks = [(0, neA), (neA, E - neA)]

    def _blk(n_, cap):
        for k in range(cap, 7, -8):
            if n_ % k == 0:
                return k
        raise ValueError(n_)

    gathers = [_make_gather(N, E, ND, _blk(n_ // NW, 128), e0, n_)
               for e0, n_ in chunks]
    scatters = [_make_scatter(half, Hp, E, ND, K, e0, n_) for e0, n_ in chunks]
    zeros_init = jnp.zeros((Hp // NS, ND), F32)

    ZD = 2 * ND + ED

    def make_msg_call(e0, n_):
        for Bee in range(4096, 7, -8):
            if n_ % Bee == 0 and e0 % Bee == 0:
                break
        blk0 = e0 // Bee
        return pl.pallas_call(
            functools.partial(_msg_body, ND=ND),
            grid=(n_ // Bee,),
            in_specs=[
                pl.BlockSpec((Bee, ND), lambda i: (i, 0)),
                pl.BlockSpec((Bee, ND), lambda i: (i, 0)),
                pl.BlockSpec((Bee, ED), lambda i: (i + blk0, 0)),
                _full((ZD, 2 * ND)),
                _full((1, 2 * ND)),
            ],
            out_specs=pl.BlockSpec((Bee, ND), lambda i: (i, 0)),
            out_shape=jax.ShapeDtypeStruct((n_, ND), F32),
        )

    msg_calls = [make_msg_call(e0, n_) for e0, n_ in chunks]
    Bea = 2560
    ea16 = pl.pallas_call(
        _ea_body,
        grid=(E // Bea,),
        in_specs=[
            pl.BlockSpec((2, Bea), lambda i: (0, i)),
            _full((2, ED)),
            _full((1, ED)),
        ],
        out_specs=pl.BlockSpec((Bea, ED), lambda i: (i, 0)),
        out_shape=jax.ShapeDtypeStruct((E, ED), F32),
    )(edge_attr.T, edge_W, edge_b.reshape(1, ED))
    bias_all = jnp.concatenate([conv_bf, conv_bs], axis=1).reshape(NL, 1,
                                                                   2 * ND)
    Wfull_all = jnp.concatenate([conv_Wf, conv_Ws], axis=2)

    part_spec = pl.BlockSpec((1, Bn, ND), lambda p, i: (i // bpc, i % bpc, 0))
    bn_call = pl.pallas_call(
        functools.partial(_bn_body, inv_n=inv_n),
        grid=(3, nb),
        in_specs=[
            part_spec,
            part_spec,
            pl.BlockSpec((Bn, ND), lambda p, i: (i, 0)),
            _full((1, ND)),
            _full((1, ND)),
        ],
        out_specs=pl.BlockSpec((Bn, ND), lambda p, i: (i, 0)),
        out_shape=jax.ShapeDtypeStruct((N, ND), F32),
        scratch_shapes=[pltpu.VMEM((1, ND), F32), pltpu.VMEM((1, ND), F32)],
    )

    for l in range(NL):
        gs_pairs = [g_k(h, dst, src) for g_k in gathers]
        msgs = [m_c(gd, gs, ea16, Wfull_all[l], bias_all[l])
                for m_c, (gd, gs) in zip(msg_calls, gs_pairs)]
        parts = [s_k(m, dst, zeros_init) for s_k, m in zip(scatters, msgs)]
        h = bn_call(parts[0], parts[1], h, conv_g[l].reshape(1, ND),
                    conv_be[l].reshape(1, ND))

    def stage(h, W, b, g, be, use_mm, Dout):
        return pl.pallas_call(
            functools.partial(_stage_body, use_mm=use_mm, Bn=Bn, inv_n=inv_n),
            grid=(3, nb),
            in_specs=[
                pl.BlockSpec((Bn, ND), lambda p, i: (i, 0)),
                _full((ND, Dout)),
                _full((1, Dout)),
                _full((1, Dout)),
                _full((1, Dout)),
            ],
            out_specs=pl.BlockSpec((Bn, Dout), lambda p, i: (i, 0)),
            out_shape=jax.ShapeDtypeStruct((N, Dout), F32),
            scratch_shapes=[
                pltpu.VMEM((N, Dout), F32),
                pltpu.VMEM((1, Dout), F32),
                pltpu.VMEM((1, Dout), F32),
            ],
        )(h, W, b.reshape(1, Dout), g.reshape(1, Dout), be.reshape(1, Dout))

    eye = jnp.eye(ND, dtype=F32)
    h = stage(h, eye, jnp.zeros((ND,), F32), sbn_g, sbn_b, False, ND)

    nstage = NFC + 1
    W_all = jnp.concatenate([fc_W.reshape(1, ND, H), head_W], axis=0)
    b_all = jnp.concatenate([fc_b.reshape(1, H), head_b], axis=0)
    g_all = jnp.concatenate([fbn_g.reshape(1, H), head_g], axis=0)
    be_all = jnp.concatenate([fbn_b.reshape(1, H), head_be], axis=0)
    svec = pl.BlockSpec((1, 1, H), lambda st, p, i: (st, 0, 0))
    out = pl.pallas_call(
        functools.partial(_tail_body, Bn=Bt, inv_n=inv_n, nstage=nstage),
        grid=(nstage, 3, ntb),
        in_specs=[
            pl.BlockSpec((Bt, ND), lambda st, p, i: (i, 0)),
            pl.BlockSpec((1, ND, H), lambda st, p, i: (st, 0, 0)),
            svec,
            svec,
            svec,
            _full((H, 1)),
            _full((1, 1)),
        ],
        out_specs=pl.BlockSpec((Bt, 1), lambda st, p, i: (i, 0)),
        out_shape=jax.ShapeDtypeStruct((N, 1), F32),
        scratch_shapes=[
            pltpu.VMEM((N, H), F32),
            pltpu.VMEM((N, H), F32),
            pltpu.VMEM((1, H), F32),
            pltpu.VMEM((1, H), F32),
        ],
    )(h, W_all, b_all.reshape(nstage, 1, H), g_all.reshape(nstage, 1, H),
      be_all.reshape(nstage, 1, H), out_W, out_b.reshape(1, 1))
    return out
